# baseline (speedup 1.0000x reference)
import functools

import jax
import jax.numpy as jnp
from jax import lax
from jax.experimental import pallas as pl
from jax.experimental.pallas import tpu as pltpu
from jax.experimental.pallas import tpu_sc as plsc

_B = 128
_F = 32768
_K = 256
_CAP = 512
_NBINS = 4096
_SHIFT = 19
_NC = 2
_NS = 16
_NW = _NC * _NS
_RPW = _B // _NW
_L = 16


def _sc_filter_body(x_hbm, out_a_hbm, out_b_hbm, rowa_v, rowb_v, hist_v,
                    ca0_v, cb0_v, ca1_v, cb1_v, sem, sem_out):
    wid = lax.axis_index("s") * _NC + lax.axis_index("c")
    zeros = jnp.zeros((_L,), jnp.int32)
    ones = jnp.ones((_L,), jnp.int32)
    pad_b = jnp.full((_L,), 0x7FFFFFFF, jnp.int32)
    iota16 = lax.iota(jnp.int32, _L)
    sub_off = (iota16 & 7) * _NBINS
    r0 = wid * _RPW
    bufs = [rowa_v, rowb_v]
    cands = [(ca0_v, cb0_v), (ca1_v, cb1_v)]
    out_cps = [None, None]

    cp = pltpu.async_copy(x_hbm.at[r0], bufs[0], sem)
    for k in range(_RPW):
        cp.wait()
        if k + 1 < _RPW:
            cp = pltpu.async_copy(
                x_hbm.at[r0 + k + 1], bufs[(k + 1) % 2], sem)
        row_v = bufs[k % 2]
        ca_v, cb_v = cands[k % 2]
        if out_cps[k % 2] is not None:
            for h in out_cps[k % 2]:
                h.wait()
            out_cps[k % 2] = None

        @plsc.parallel_loop(0, 8 * _NBINS // _L, unroll=8)
        def _(i):
            hist_v[pl.ds(i * _L, _L)] = zeros

        @plsc.parallel_loop(0, _CAP // _L, unroll=8)
        def _(i):
            ca_v[pl.ds(i * _L, _L)] = zeros
            cb_v[pl.ds(i * _L, _L)] = pad_b

        @plsc.parallel_loop(0, _F // _L, unroll=16, carry=zeros)
        def mx_v(i, mx):
            v = row_v[pl.ds(i * _L, _L)]
            u = plsc.bitcast(v, jnp.uint32)
            bin_i = plsc.bitcast((u << 1) >> (_SHIFT + 1), jnp.int32)
            plsc.addupdate_scatter(hist_v, [bin_i + sub_off], ones)
            return jnp.maximum(mx, bin_i)

        def scond(st):
            i, acc, done = st
            return jnp.logical_not(done)

        def _chunk_sums(i):
            s = hist_v[pl.ds(i * _L, _L)]
            for j in range(1, 8):
                s = s + hist_v[pl.ds(j * _NBINS + i * _L, _L)]
            return s

        def sbody(st):
            i, acc, done = st
            tot = jnp.sum(_chunk_sums(i))
            crosses = (acc + tot) >= _K
            return (jnp.where(crosses, i, i - 1),
                    jnp.where(crosses, acc, acc + tot),
                    crosses)

        ci, acc_above, _ = lax.while_loop(
            scond, sbody, (jnp.max(mx_v) >> 4, jnp.int32(0), False))

        h16 = _chunk_sums(ci)
        suff = lax.rev(jnp.cumsum(lax.rev(h16, (0,))), (0,)) + acc_above
        lane = jnp.sum((suff >= _K).astype(jnp.int32)) - 1
        t_key = (ci * _L + lane) << _SHIFT

        t_key_v = jnp.full((_L,), 0, jnp.int32) + t_key

        @plsc.parallel_loop(0, _F // _L, unroll=8, carry=zeros)
        def off_v(i, off):
            v = row_v[pl.ds(i * _L, _L)]
            bits = plsc.bitcast(v, jnp.int32)
            key = bits & 0x7FFFFFFF
            m = key >= t_key_v
            mi = m.astype(jnp.int32)
            pos = jnp.minimum(jnp.cumsum(mi) + (off - 1), _CAP - 1)
            sign = lax.shift_right_logical(bits, 31)
            bval = ((iota16 + i * _L) << 1) | sign
            plsc.store_scatter(ca_v, [pos], key, mask=m)
            plsc.store_scatter(cb_v, [pos], bval, mask=m)
            return off + plsc.all_reduce_population_count(m)

        del off_v
        out_cps[k % 2] = [
            pltpu.async_copy(ca_v, out_a_hbm.at[r0 + k], sem_out),
            pltpu.async_copy(cb_v, out_b_hbm.at[r0 + k], sem_out),
        ]

    for cps in out_cps:
        if cps is not None:
            for h in cps:
                h.wait()


@functools.cache
def _get_sc_filter():
    return pl.kernel(
        _sc_filter_body,
        mesh=plsc.VectorSubcoreMesh(core_axis_name="c", subcore_axis_name="s"),
        compiler_params=pltpu.CompilerParams(needs_layout_passes=False),
        out_type=[
            jax.ShapeDtypeStruct((_B, _CAP), jnp.int32),
            jax.ShapeDtypeStruct((_B, _CAP), jnp.int32),
        ],
        scratch_types=[
            pltpu.VMEM((_F,), jnp.float32),
            pltpu.VMEM((_F,), jnp.float32),
            pltpu.VMEM((8 * _NBINS,), jnp.int32),
            pltpu.VMEM((_CAP,), jnp.int32),
            pltpu.VMEM((_CAP,), jnp.int32),
            pltpu.VMEM((_CAP,), jnp.int32),
            pltpu.VMEM((_CAP,), jnp.int32),
            pltpu.SemaphoreType.DMA,
            pltpu.SemaphoreType.DMA,
        ],
    )


def _tc_sort_body(a_ref, b_ref, o_ref):
    a = a_ref[...].T
    b = b_ref[...].T
    iota = lax.broadcasted_iota(jnp.int32, (_CAP, 1), 0)
    k = 2
    while k <= _CAP:
        j = k // 2
        while j >= 1:
            g = _CAP // (2 * j)
            ar = a.reshape(g, 2, j, _B)
            br = b.reshape(g, 2, j, _B)
            ap = jnp.concatenate([ar[:, 1:2], ar[:, 0:1]], axis=1).reshape(_CAP, _B)
            bp = jnp.concatenate([br[:, 1:2], br[:, 0:1]], axis=1).reshape(_CAP, _B)
            up = (iota & k) == 0
            is_lower = (iota & j) == 0
            keep_min = up == is_lower
            mine_is_min = jnp.logical_or(
                a > ap, jnp.logical_and(a == ap, b < bp))
            take_mine = mine_is_min == keep_min
            a = jnp.where(take_mine, a, ap)
            b = jnp.where(take_mine, b, bp)
            j //= 2
        k *= 2
    val_bits = a[0:_K] | (b[0:_K] << 31)
    o_ref[...] = lax.bitcast_convert_type(val_bits, jnp.float32).T


_tc_sort = pl.pallas_call(
    _tc_sort_body,
    out_shape=jax.ShapeDtypeStruct((_B, _K), jnp.float32),
)


def kernel(x):
    cand_a, cand_b = _get_sc_filter()(x)
    return _tc_sort(cand_a, cand_b)[..., None]


# speedup vs baseline: 1.0858x; 1.0858x over previous
import functools

import jax
import jax.numpy as jnp
from jax import lax
from jax.experimental import pallas as pl
from jax.experimental.pallas import tpu as pltpu
from jax.experimental.pallas import tpu_sc as plsc

_B = 128
_F = 32768
_K = 256
_CAP = 512
_NBINS = 4096
_SHIFT = 19
_NC = 2
_NS = 16
_NW = _NC * _NS
_RPW = _B // _NW
_L = 16


def _sc_filter_body(x_hbm, out_a_hbm, out_b_hbm, rowa_v, rowb_v, hist_v,
                    ca0_v, cb0_v, ca1_v, cb1_v, sem, sem_out):
    wid = lax.axis_index("s") * _NC + lax.axis_index("c")
    zeros = jnp.zeros((_L,), jnp.int32)
    ones = jnp.ones((_L,), jnp.int32)
    pad_b = jnp.full((_L,), 0x7FFFFFFF, jnp.int32)
    iota16 = lax.iota(jnp.int32, _L)
    r0 = wid * _RPW
    bufs = [rowa_v, rowb_v]
    cands = [(ca0_v, cb0_v), (ca1_v, cb1_v)]
    out_cps = [None, None]

    cp = pltpu.async_copy(x_hbm.at[r0], bufs[0], sem)
    for k in range(_RPW):
        cp.wait()
        if k + 1 < _RPW:
            cp = pltpu.async_copy(
                x_hbm.at[r0 + k + 1], bufs[(k + 1) % 2], sem)
        row_v = bufs[k % 2]
        ca_v, cb_v = cands[k % 2]
        if out_cps[k % 2] is not None:
            for h in out_cps[k % 2]:
                h.wait()
            out_cps[k % 2] = None

        @plsc.parallel_loop(0, _NBINS // _L, unroll=8)
        def _(i):
            hist_v[pl.ds(i * _L, _L)] = zeros

        @plsc.parallel_loop(0, _CAP // _L, unroll=8)
        def _(i):
            ca_v[pl.ds(i * _L, _L)] = zeros
            cb_v[pl.ds(i * _L, _L)] = pad_b

        @plsc.parallel_loop(0, _F // _L, unroll=16, carry=zeros)
        def mx_v(i, mx):
            v = row_v[pl.ds(i * _L, _L)]
            u = plsc.bitcast(v, jnp.uint32)
            bin_i = plsc.bitcast((u << 1) >> (_SHIFT + 1), jnp.int32)
            plsc.addupdate_scatter(hist_v, [bin_i], ones)
            return jnp.maximum(mx, bin_i)

        def scond(st):
            i, acc, done = st
            return jnp.logical_not(done)

        def sbody(st):
            i, acc, done = st
            tot = jnp.sum(hist_v[pl.ds(i * _L, _L)])
            crosses = (acc + tot) >= _K
            return (jnp.where(crosses, i, i - 1),
                    jnp.where(crosses, acc, acc + tot),
                    crosses)

        ci, acc_above, _ = lax.while_loop(
            scond, sbody, (jnp.max(mx_v) >> 4, jnp.int32(0), False))

        h16 = hist_v[pl.ds(ci * _L, _L)]
        suff = lax.rev(jnp.cumsum(lax.rev(h16, (0,))), (0,)) + acc_above
        lane = jnp.sum((suff >= _K).astype(jnp.int32)) - 1
        t_key = (ci * _L + lane) << _SHIFT

        t_key_v = jnp.full((_L,), 0, jnp.int32) + t_key

        @plsc.parallel_loop(0, _F // _L, unroll=8, carry=zeros)
        def off_v(i, off):
            v = row_v[pl.ds(i * _L, _L)]
            bits = plsc.bitcast(v, jnp.int32)
            key = bits & 0x7FFFFFFF
            m = key >= t_key_v
            mi = m.astype(jnp.int32)
            pos = jnp.minimum(jnp.cumsum(mi) + (off - 1), _CAP - 1)
            sign = lax.shift_right_logical(bits, 31)
            bval = ((iota16 + i * _L) << 1) | sign
            plsc.store_scatter(ca_v, [pos], key, mask=m)
            plsc.store_scatter(cb_v, [pos], bval, mask=m)
            return off + plsc.all_reduce_population_count(m)

        del off_v
        out_cps[k % 2] = [
            pltpu.async_copy(ca_v, out_a_hbm.at[r0 + k], sem_out),
            pltpu.async_copy(cb_v, out_b_hbm.at[r0 + k], sem_out),
        ]

    for cps in out_cps:
        if cps is not None:
            for h in cps:
                h.wait()


@functools.cache
def _get_sc_filter():
    return pl.kernel(
        _sc_filter_body,
        mesh=plsc.VectorSubcoreMesh(core_axis_name="c", subcore_axis_name="s"),
        compiler_params=pltpu.CompilerParams(needs_layout_passes=False),
        out_type=[
            jax.ShapeDtypeStruct((_B, _CAP), jnp.int32),
            jax.ShapeDtypeStruct((_B, _CAP), jnp.int32),
        ],
        scratch_types=[
            pltpu.VMEM((_F,), jnp.float32),
            pltpu.VMEM((_F,), jnp.float32),
            pltpu.VMEM((_NBINS,), jnp.int32),
            pltpu.VMEM((_CAP,), jnp.int32),
            pltpu.VMEM((_CAP,), jnp.int32),
            pltpu.VMEM((_CAP,), jnp.int32),
            pltpu.VMEM((_CAP,), jnp.int32),
            pltpu.SemaphoreType.DMA,
            pltpu.SemaphoreType.DMA,
        ],
    )


def _tc_sort_body(a_ref, b_ref, o_ref):
    a = a_ref[...].T
    b = b_ref[...].T
    iota = lax.broadcasted_iota(jnp.int32, (_CAP, 1), 0)
    k = 2
    while k <= _CAP:
        j = k // 2
        while j >= 1:
            g = _CAP // (2 * j)
            ar = a.reshape(g, 2, j, _B)
            br = b.reshape(g, 2, j, _B)
            ap = jnp.concatenate([ar[:, 1:2], ar[:, 0:1]], axis=1).reshape(_CAP, _B)
            bp = jnp.concatenate([br[:, 1:2], br[:, 0:1]], axis=1).reshape(_CAP, _B)
            up = (iota & k) == 0
            is_lower = (iota & j) == 0
            keep_min = up == is_lower
            mine_is_min = jnp.logical_or(
                a > ap, jnp.logical_and(a == ap, b < bp))
            take_mine = mine_is_min == keep_min
            a = jnp.where(take_mine, a, ap)
            b = jnp.where(take_mine, b, bp)
            j //= 2
        k *= 2
    val_bits = a[0:_K] | (b[0:_K] << 31)
    o_ref[...] = lax.bitcast_convert_type(val_bits, jnp.float32).T


_tc_sort = pl.pallas_call(
    _tc_sort_body,
    out_shape=jax.ShapeDtypeStruct((_B, _K), jnp.float32),
)


def kernel(x):
    cand_a, cand_b = _get_sc_filter()(x)
    return _tc_sort(cand_a, cand_b)[..., None]
